# baseline (speedup 1.0000x reference)
import functools

import jax
import jax.numpy as jnp
from jax import lax
from jax.experimental import pallas as pl
from jax.experimental.pallas import tpu as pltpu
from jax.experimental.pallas import tpu_sc as plsc

B = 4096
C = 50
V = 1000000
A = 1000
D = 16

NC, NS = 2, 16
NW = NC * NS
SAMPLES_PER_W = B // NW
CHUNK_SAMPLES = 2
CHUNK_ROWS = CHUNK_SAMPLES * C
NCHUNK = SAMPLES_PER_W // CHUNK_SAMPLES
NBUF = 2


def _tree_sum(vals):
    while len(vals) > 1:
        vals = [vals[i] + vals[i + 1] if i + 1 < len(vals) else vals[i]
                for i in range(0, len(vals), 2)]
    return vals[0]


def _sc_bag_kernel(obs2d_hbm, table_hbm, out_hbm, idx_v, rows_v, enc_v,
                   sem0, sem1):
    wid = lax.axis_index("s") * NC + lax.axis_index("c")
    pltpu.sync_copy(obs2d_hbm.at[pl.ds(wid * NCHUNK, NCHUNK)], idx_v)

    def fire(j, buf):
        pltpu.async_copy(table_hbm.at[idx_v.at[j]], rows_v.at[buf],
                         sem0 if buf == 0 else sem1)

    def wait(j, buf):
        pltpu.make_async_copy(table_hbm.at[idx_v.at[j]], rows_v.at[buf],
                              sem0 if buf == 0 else sem1).wait()

    fire(0, 0)
    fire(1, 1)

    def step(j2, _):
        for p in range(NBUF):
            j = j2 + p
            wait(j, p)
            rows = rows_v.at[p]
            for s in range(CHUNK_SAMPLES):
                acc = _tree_sum([rows[s * C + c, :] for c in range(C)])
                enc_v[j * CHUNK_SAMPLES + s, :] = acc
            nj = j + NBUF

            @pl.when(nj < NCHUNK)
            def _():
                fire(nj, p)
        return ()

    lax.fori_loop(0, NCHUNK // NBUF,
                  lambda i, c: step(i * NBUF, c), (), unroll=False)
    pltpu.sync_copy(enc_v, out_hbm.at[pl.ds(wid * SAMPLES_PER_W,
                                            SAMPLES_PER_W)])


@jax.jit
def _sc_bag(obs2d, table):
    mesh = plsc.VectorSubcoreMesh(core_axis_name="c", subcore_axis_name="s")
    return pl.kernel(
        _sc_bag_kernel,
        out_type=jax.ShapeDtypeStruct((B, D), jnp.float32),
        mesh=mesh,
        scratch_types=[
            pltpu.VMEM((NCHUNK, CHUNK_ROWS), jnp.int32),
            pltpu.VMEM((NBUF, CHUNK_ROWS, D), jnp.float32),
            pltpu.VMEM((SAMPLES_PER_W, D), jnp.float32),
            pltpu.SemaphoreType.DMA,
            pltpu.SemaphoreType.DMA,
        ],
        compiler_params=pltpu.CompilerParams(use_tc_tiling_on_sc=False),
    )(obs2d, table)


def _tc_combine_kernel(enc_ref, act_ref, table_ref, w_ref, b_ref, out_ref):
    w1 = w_ref[0:1, 0:D]
    w2 = w_ref[0:1, D:2 * D]
    actproj = jnp.dot(table_ref[...], w2.T,
                      preferred_element_type=jnp.float32)
    y_act = jnp.dot(act_ref[...], actproj,
                    preferred_element_type=jnp.float32)
    y_obs = jnp.dot(enc_ref[...], w1.T,
                    preferred_element_type=jnp.float32)
    out_ref[...] = y_obs * (1.0 / C) + y_act * (1.0 / A) + b_ref[0]


@jax.jit
def _tc_combine(enc, actions2d, act_table, W, b):
    bm = 512
    grid = (B // bm,)
    return pl.pallas_call(
        _tc_combine_kernel,
        grid=grid,
        in_specs=[
            pl.BlockSpec((bm, D), lambda i: (i, 0)),
            pl.BlockSpec((bm, A), lambda i: (i, 0)),
            pl.BlockSpec((A, D), lambda i: (0, 0)),
            pl.BlockSpec((1, 2 * D), lambda i: (0, 0)),
            pl.BlockSpec(memory_space=pltpu.SMEM),
        ],
        out_specs=pl.BlockSpec((bm, 1), lambda i: (i, 0)),
        out_shape=jax.ShapeDtypeStruct((B, 1), jnp.float32),
    )(enc, actions2d, act_table, W, b)


def kernel(observation, actions, obs_table, act_table, W, b):
    obs2d = observation.astype(jnp.int32).reshape(B * C // CHUNK_ROWS,
                                                  CHUNK_ROWS)
    enc = _sc_bag(obs2d, obs_table)
    actions2d = actions.reshape(B, A)
    return _tc_combine(enc, actions2d, act_table, W, b)
